# Initial kernel scaffold; baseline (speedup 1.0000x reference)
import jax
import jax.numpy as jnp
from jax.experimental import pallas as pl


def kernel(sentence_embeddings, text_embeddings, structural_embeddings):
    raise NotImplementedError("write your pallas kernel here")


# speedup vs baseline: 1.3382x; 1.3382x over previous
import jax
import jax.numpy as jnp
from jax.experimental import pallas as pl
from jax.experimental.pallas import tpu as pltpu
from jax.experimental.pallas import tpu_sc as plsc

B = 4096
D = 384
N = 100000
S = 256
K = 3
W_STRUCT = 0.3
REL_THRESH = 0.1

BK = 512
BQ = 1024
NQ = B // BQ
NK = (N + BK - 1) // BK
NEG = -1.0e30


def _merge_top3(av, ai, bv, bi):
    out_v, out_i = [], []
    c = av[0] >= bv[0]
    out_v.append(jnp.where(c, av[0], bv[0]))
    out_i.append(jnp.where(c, ai[0], bi[0]))
    a1v = [jnp.where(c, av[1], av[0]), jnp.where(c, av[2], av[1])]
    a1i = [jnp.where(c, ai[1], ai[0]), jnp.where(c, ai[2], ai[1])]
    b1v = [jnp.where(c, bv[0], bv[1]), jnp.where(c, bv[1], bv[2])]
    b1i = [jnp.where(c, bi[0], bi[1]), jnp.where(c, bi[1], bi[2])]
    c = a1v[0] >= b1v[0]
    out_v.append(jnp.where(c, a1v[0], b1v[0]))
    out_i.append(jnp.where(c, a1i[0], b1i[0]))
    a2v = jnp.where(c, a1v[1], a1v[0])
    a2i = jnp.where(c, a1i[1], a1i[0])
    b2v = jnp.where(c, b1v[0], b1v[1])
    b2i = jnp.where(c, b1i[0], b1i[1])
    c = a2v >= b2v
    out_v.append(jnp.where(c, a2v, b2v))
    out_i.append(jnp.where(c, a2i, b2i))
    return out_v, out_i


def _topk_kernel(q_ref, kt_ref, w_ref, idx_ref, qn_ref,
                 rv0, rv1, rv2, ri0, ri1, ri2):
    kblk = pl.program_id(1)

    @pl.when(kblk == 0)
    def _():
        q = q_ref[...]
        qn = q / (jnp.sqrt(jnp.sum(q * q, axis=1, keepdims=True)) + 1e-12)
        qn_ref[...] = qn.astype(jnp.bfloat16)

    kt = kt_ref[...]
    norm = jnp.sqrt(jnp.sum(kt * kt, axis=0, keepdims=True))
    kn = (kt / (norm + 1e-12)).astype(jnp.bfloat16)
    s = jax.lax.dot_general(qn_ref[...], kn, (((1,), (0,)), ((), ())),
                            preferred_element_type=jnp.float32)

    liota = jax.lax.broadcasted_iota(jnp.int32, (BQ, BK), 1)
    gcol = kblk * BK + liota
    s = jnp.where(gcol < N, s, NEG)

    bv, bi = [], []
    for j in range(K):
        m = jnp.max(s, axis=1, keepdims=True)
        p = jnp.min(jnp.where(s == m, liota, BK), axis=1, keepdims=True)
        bv.append(m)
        bi.append(p + kblk * BK)
        if j < K - 1:
            s = jnp.where(liota == p, NEG, s)

    def _store(vs, is_):
        rv0[...], rv1[...], rv2[...] = vs[0], vs[1], vs[2]
        ri0[...], ri1[...], ri2[...] = is_[0], is_[1], is_[2]

    @pl.when(kblk == 0)
    def _():
        _store(bv, bi)

    @pl.when(kblk > 0)
    def _():
        mv, mi = _merge_top3([rv0[...], rv1[...], rv2[...]],
                             [ri0[...], ri1[...], ri2[...]], bv, bi)
        _store(mv, mi)

    @pl.when(kblk == NK - 1)
    def _():
        vs = [rv0[...], rv1[...], rv2[...]]
        is_ = [ri0[...], ri1[...], ri2[...]]
        ms = [jnp.where(v > REL_THRESH, v, 0.0) for v in vs]
        ssum = ms[0] + ms[1] + ms[2] + 1e-8
        ws = [m / ssum for m in ms]
        w_ref[...] = jnp.concatenate(ws, axis=1)
        idx_ref[...] = jnp.concatenate(is_, axis=1)


def _topk(q, kt):
    return pl.pallas_call(
        _topk_kernel,
        grid=(NQ, NK),
        in_specs=[
            pl.BlockSpec((BQ, D), lambda i, k: (i, 0)),
            pl.BlockSpec((D, BK), lambda i, k: (0, k)),
        ],
        out_specs=[
            pl.BlockSpec((BQ, K), lambda i, k: (i, 0)),
            pl.BlockSpec((BQ, K), lambda i, k: (i, 0)),
        ],
        out_shape=[
            jax.ShapeDtypeStruct((B, K), jnp.float32),
            jax.ShapeDtypeStruct((B, K), jnp.int32),
        ],
        scratch_shapes=[
            pltpu.VMEM((BQ, D), jnp.bfloat16),
            pltpu.VMEM((BQ, 1), jnp.float32),
            pltpu.VMEM((BQ, 1), jnp.float32),
            pltpu.VMEM((BQ, 1), jnp.float32),
            pltpu.VMEM((BQ, 1), jnp.int32),
            pltpu.VMEM((BQ, 1), jnp.int32),
            pltpu.VMEM((BQ, 1), jnp.int32),
        ],
    )(q, kt)


_GATHER_WIN = 128


def _sc_gather(struct, idx_flat):
    n_idx = B * K

    @pl.kernel(
        out_type=jax.ShapeDtypeStruct((n_idx, S), jnp.float32),
        mesh=plsc.VectorSubcoreMesh(core_axis_name="core",
                                    subcore_axis_name="subcore"),
    )
    def _gather(x_hbm, i_hbm, o_hbm):
        def body(i_vmem, o_vmem):
            pltpu.sync_copy(x_hbm.at[i_vmem.at[0]], o_vmem)

        pltpu.emit_pipeline(
            body,
            grid=(n_idx // _GATHER_WIN,),
            in_specs=[pl.BlockSpec((1, _GATHER_WIN), lambda i: (0, i))],
            out_specs=[pl.BlockSpec((_GATHER_WIN, S), lambda i: (i, 0))],
            core_axis_name="subcore",
            dimension_semantics=(pltpu.PARALLEL,),
        )(i_hbm, o_hbm)

    return _gather(struct, idx_flat)


def _combine_kernel(w_ref, r0_ref, r1_ref, r2_ref, o_ref):
    w = w_ref[...]
    acc = (w[:, 0:1] * r0_ref[...] + w[:, 1:2] * r1_ref[...]
           + w[:, 2:3] * r2_ref[...])
    o_ref[...] = W_STRUCT * acc


def _combine(w, r0, r1, r2):
    bq = 1024
    return pl.pallas_call(
        _combine_kernel,
        grid=(B // bq,),
        in_specs=[
            pl.BlockSpec((bq, K), lambda i: (i, 0)),
            pl.BlockSpec((bq, S), lambda i: (i, 0)),
            pl.BlockSpec((bq, S), lambda i: (i, 0)),
            pl.BlockSpec((bq, S), lambda i: (i, 0)),
        ],
        out_specs=pl.BlockSpec((bq, S), lambda i: (i, 0)),
        out_shape=jax.ShapeDtypeStruct((B, S), jnp.float32),
    )(w, r0, r1, r2)


def kernel(sentence_embeddings, text_embeddings, structural_embeddings):
    kt = text_embeddings.T
    w, idx = _topk(sentence_embeddings, kt)
    idx_flat = idx.reshape(1, B * K)
    retrieved = _sc_gather(structural_embeddings, idx_flat)
    r = retrieved.reshape(B, K, S)
    struct_ctx = _combine(w, r[:, 0, :], r[:, 1, :], r[:, 2, :])
    return jnp.concatenate([sentence_embeddings, struct_ctx], axis=-1)


# speedup vs baseline: 1.7487x; 1.3068x over previous
import jax
import jax.numpy as jnp
from jax.experimental import pallas as pl
from jax.experimental.pallas import tpu as pltpu
from jax.experimental.pallas import tpu_sc as plsc

B = 4096
D = 384
N = 100000
S = 256
K = 3
W_STRUCT = 0.3
REL_THRESH = 0.1

BK = 512
BQ = 1024
NQ = B // BQ
NK = (N + BK - 1) // BK
NKP = NK * BK
NEG = -1.0e30
PACK = (BQ // 128, 128)


def _norm_kernel(k_ref, o_ref):
    kb = k_ref[...]
    nrm = jnp.sqrt(jnp.sum(kb * kb, axis=1, keepdims=True))
    o_ref[...] = (kb / (nrm + 1e-12)).astype(jnp.bfloat16)


_BN = 4000


def _normalize_keys(te):
    return pl.pallas_call(
        _norm_kernel,
        grid=(N // _BN,),
        in_specs=[pl.BlockSpec((_BN, D), lambda i: (i, 0))],
        out_specs=pl.BlockSpec((_BN, D), lambda i: (i, 0)),
        out_shape=jax.ShapeDtypeStruct((N, D), jnp.bfloat16),
    )(te)


def _merge_top3(av, ai, bv, bi):
    out_v, out_i = [], []
    c = av[0] >= bv[0]
    out_v.append(jnp.where(c, av[0], bv[0]))
    out_i.append(jnp.where(c, ai[0], bi[0]))
    a1v = [jnp.where(c, av[1], av[0]), jnp.where(c, av[2], av[1])]
    a1i = [jnp.where(c, ai[1], ai[0]), jnp.where(c, ai[2], ai[1])]
    b1v = [jnp.where(c, bv[0], bv[1]), jnp.where(c, bv[1], bv[2])]
    b1i = [jnp.where(c, bi[0], bi[1]), jnp.where(c, bi[1], bi[2])]
    c = a1v[0] >= b1v[0]
    out_v.append(jnp.where(c, a1v[0], b1v[0]))
    out_i.append(jnp.where(c, a1i[0], b1i[0]))
    a2v = jnp.where(c, a1v[1], a1v[0])
    a2i = jnp.where(c, a1i[1], a1i[0])
    b2v = jnp.where(c, b1v[0], b1v[1])
    b2i = jnp.where(c, b1i[0], b1i[1])
    c = a2v >= b2v
    out_v.append(jnp.where(c, a2v, b2v))
    out_i.append(jnp.where(c, a2i, b2i))
    return out_v, out_i


def _pack(x):
    return x.reshape(PACK)


def _topk_kernel(q_ref, kn_ref, w_ref, idx_ref, qn_ref,
                 rv0, rv1, rv2, ri0, ri1, ri2):
    kblk = pl.program_id(1)

    @pl.when(kblk == 0)
    def _():
        q = q_ref[...]
        qn = q / (jnp.sqrt(jnp.sum(q * q, axis=1, keepdims=True)) + 1e-12)
        qn_ref[...] = qn.astype(jnp.bfloat16)

    kn = kn_ref[...]
    s = jax.lax.dot_general(qn_ref[...], kn, (((1,), (1,)), ((), ())),
                            preferred_element_type=jnp.float32)

    liota = jax.lax.broadcasted_iota(jnp.int32, (BQ, BK), 1)
    gcol = kblk * BK + liota
    s = jnp.where(gcol < N, s, NEG)

    m0 = jnp.max(s, axis=1, keepdims=True)
    s1 = jnp.where(s == m0, NEG, s)
    m1 = jnp.max(s1, axis=1, keepdims=True)
    s2 = jnp.where(s1 == m1, NEG, s1)
    m2 = jnp.max(s2, axis=1, keepdims=True)
    p0 = jnp.min(jnp.where(s == m0, liota, BK), axis=1, keepdims=True)
    p1 = jnp.min(jnp.where(s == m1, liota, BK), axis=1, keepdims=True)
    p2 = jnp.min(jnp.where(s == m2, liota, BK), axis=1, keepdims=True)

    koff = kblk * BK
    bv = [_pack(m0), _pack(m1), _pack(m2)]
    bi = [_pack(p0) + koff, _pack(p1) + koff, _pack(p2) + koff]

    def _store(vs, is_):
        rv0[...], rv1[...], rv2[...] = vs[0], vs[1], vs[2]
        ri0[...], ri1[...], ri2[...] = is_[0], is_[1], is_[2]

    @pl.when(kblk == 0)
    def _():
        _store(bv, bi)

    @pl.when(kblk > 0)
    def _():
        mv, mi = _merge_top3([rv0[...], rv1[...], rv2[...]],
                             [ri0[...], ri1[...], ri2[...]], bv, bi)
        _store(mv, mi)

    @pl.when(kblk == NK - 1)
    def _():
        vs = [rv0[...], rv1[...], rv2[...]]
        is_ = [ri0[...], ri1[...], ri2[...]]
        ms = [jnp.where(v > REL_THRESH, v, 0.0) for v in vs]
        ssum = ms[0] + ms[1] + ms[2] + 1e-8
        for j in range(K):
            w_ref[j, :, :] = ms[j] / ssum
            idx_ref[j, :, :] = is_[j]


def _topk(q, kn16):
    return pl.pallas_call(
        _topk_kernel,
        grid=(NQ, NK),
        in_specs=[
            pl.BlockSpec((BQ, D), lambda i, k: (i, 0)),
            pl.BlockSpec((BK, D), lambda i, k: (k, 0)),
        ],
        out_specs=[
            pl.BlockSpec((K, PACK[0], 128), lambda i, k: (0, i, 0)),
            pl.BlockSpec((K, PACK[0], 128), lambda i, k: (0, i, 0)),
        ],
        out_shape=[
            jax.ShapeDtypeStruct((K, B // 128, 128), jnp.float32),
            jax.ShapeDtypeStruct((K, B // 128, 128), jnp.int32),
        ],
        scratch_shapes=[
            pltpu.VMEM((BQ, D), jnp.bfloat16),
            pltpu.VMEM(PACK, jnp.float32),
            pltpu.VMEM(PACK, jnp.float32),
            pltpu.VMEM(PACK, jnp.float32),
            pltpu.VMEM(PACK, jnp.int32),
            pltpu.VMEM(PACK, jnp.int32),
            pltpu.VMEM(PACK, jnp.int32),
        ],
    )(q, kn16)


_GATHER_WIN = 128


def _sc_gather(struct, idx_flat):
    n_idx = B * K

    @pl.kernel(
        out_type=jax.ShapeDtypeStruct((n_idx, S), jnp.float32),
        mesh=plsc.VectorSubcoreMesh(core_axis_name="core",
                                    subcore_axis_name="subcore"),
    )
    def _gather(x_hbm, i_hbm, o_hbm):
        def body(i_vmem, o_vmem):
            pltpu.sync_copy(x_hbm.at[i_vmem.at[0]], o_vmem)

        pltpu.emit_pipeline(
            body,
            grid=(n_idx // _GATHER_WIN,),
            in_specs=[pl.BlockSpec((1, _GATHER_WIN), lambda i: (0, i))],
            out_specs=[pl.BlockSpec((_GATHER_WIN, S), lambda i: (i, 0))],
            core_axis_name="subcore",
            dimension_semantics=(pltpu.PARALLEL,),
        )(i_hbm, o_hbm)

    return _gather(struct, idx_flat)


def _combine_kernel(w_ref, r0_ref, r1_ref, r2_ref, o_ref):
    w = w_ref[...]
    acc = (w[:, 0:1] * r0_ref[...] + w[:, 1:2] * r1_ref[...]
           + w[:, 2:3] * r2_ref[...])
    o_ref[...] = W_STRUCT * acc


def _combine(w, r0, r1, r2):
    bq = 1024
    return pl.pallas_call(
        _combine_kernel,
        grid=(B // bq,),
        in_specs=[
            pl.BlockSpec((bq, K), lambda i: (i, 0)),
            pl.BlockSpec((bq, S), lambda i: (i, 0)),
            pl.BlockSpec((bq, S), lambda i: (i, 0)),
            pl.BlockSpec((bq, S), lambda i: (i, 0)),
        ],
        out_specs=pl.BlockSpec((bq, S), lambda i: (i, 0)),
        out_shape=jax.ShapeDtypeStruct((B, S), jnp.float32),
    )(w, r0, r1, r2)


def kernel(sentence_embeddings, text_embeddings, structural_embeddings):
    kn16 = _normalize_keys(text_embeddings)
    wp, idxp = _topk(sentence_embeddings, kn16)
    idx_flat = idxp.reshape(1, K * B)
    retrieved = _sc_gather(structural_embeddings, idx_flat)
    r = retrieved.reshape(K, B, S)
    w = jnp.concatenate([wp[j].reshape(B, 1) for j in range(K)], axis=1)
    struct_ctx = _combine(w, r[0], r[1], r[2])
    return jnp.concatenate([sentence_embeddings, struct_ctx], axis=-1)


# speedup vs baseline: 3.4208x; 1.9562x over previous
import jax
import jax.numpy as jnp
from jax.experimental import pallas as pl
from jax.experimental.pallas import tpu as pltpu
from jax.experimental.pallas import tpu_sc as plsc

B = 4096
D = 384
N = 100000
S = 256
K = 3
W_STRUCT = 0.3
REL_THRESH = 0.1

BK = 1024
BQ = 1024
NQ = B // BQ
NK = (N + BK - 1) // BK
NG = BK // 128
NEG = -1.0e30
PACK = (BQ // 128, 128)


def _norm_kernel(k_ref, o_ref):
    kb = k_ref[...]
    nrm = jnp.sqrt(jnp.sum(kb * kb, axis=1, keepdims=True))
    o_ref[...] = (kb / (nrm + 1e-12)).astype(jnp.bfloat16)


_BN = 4000


def _normalize_keys(te):
    return pl.pallas_call(
        _norm_kernel,
        grid=(N // _BN,),
        in_specs=[pl.BlockSpec((_BN, D), lambda i: (i, 0))],
        out_specs=pl.BlockSpec((_BN, D), lambda i: (i, 0)),
        out_shape=jax.ShapeDtypeStruct((N, D), jnp.bfloat16),
    )(te)


def _merge_top3(av, ai, bv, bi):
    out_v, out_i = [], []
    c = av[0] >= bv[0]
    out_v.append(jnp.where(c, av[0], bv[0]))
    out_i.append(jnp.where(c, ai[0], bi[0]))
    a1v = [jnp.where(c, av[1], av[0]), jnp.where(c, av[2], av[1])]
    a1i = [jnp.where(c, ai[1], ai[0]), jnp.where(c, ai[2], ai[1])]
    b1v = [jnp.where(c, bv[0], bv[1]), jnp.where(c, bv[1], bv[2])]
    b1i = [jnp.where(c, bi[0], bi[1]), jnp.where(c, bi[1], bi[2])]
    c = a1v[0] >= b1v[0]
    out_v.append(jnp.where(c, a1v[0], b1v[0]))
    out_i.append(jnp.where(c, a1i[0], b1i[0]))
    a2v = jnp.where(c, a1v[1], a1v[0])
    a2i = jnp.where(c, a1i[1], a1i[0])
    b2v = jnp.where(c, b1v[0], b1v[1])
    b2i = jnp.where(c, b1i[0], b1i[1])
    c = a2v >= b2v
    out_v.append(jnp.where(c, a2v, b2v))
    out_i.append(jnp.where(c, a2i, b2i))
    return out_v, out_i


def _pack(x):
    return x.reshape(PACK)


def _topk_kernel(q_ref, kn_ref, w_ref, idx_ref, qn_ref,
                 t0_ref, t1_ref, t2_ref, g0_ref, g1_ref, g2_ref):
    kblk = pl.program_id(1)

    @pl.when(kblk == 0)
    def _():
        q = q_ref[...]
        qn = q / (jnp.sqrt(jnp.sum(q * q, axis=1, keepdims=True)) + 1e-12)
        qn_ref[...] = qn.astype(jnp.bfloat16)
        t0_ref[...] = jnp.full((BQ, 128), NEG, jnp.float32)
        t1_ref[...] = jnp.full((BQ, 128), NEG, jnp.float32)
        t2_ref[...] = jnp.full((BQ, 128), NEG, jnp.float32)
        g0_ref[...] = jnp.zeros((BQ, 128), jnp.int32)
        g1_ref[...] = jnp.zeros((BQ, 128), jnp.int32)
        g2_ref[...] = jnp.zeros((BQ, 128), jnp.int32)

    kn = kn_ref[...]
    s = jax.lax.dot_general(qn_ref[...], kn, (((1,), (1,)), ((), ())),
                            preferred_element_type=jnp.float32)

    liota = jax.lax.broadcasted_iota(jnp.int32, (BQ, 128), 1)

    t0, t1, t2 = t0_ref[...], t1_ref[...], t2_ref[...]
    g0, g1, g2 = g0_ref[...], g1_ref[...], g2_ref[...]
    for g in range(NG):
        x = s[:, g * 128:(g + 1) * 128]
        thr = N - kblk * BK - g * 128
        x = jnp.where(liota < thr, x, NEG)
        gid = kblk * NG + g
        c0 = x > t0
        nx = jnp.where(c0, t0, x)
        ng = jnp.where(c0, g0, gid)
        t0 = jnp.where(c0, x, t0)
        g0 = jnp.where(c0, gid, g0)
        c1 = nx > t1
        nx2 = jnp.where(c1, t1, nx)
        ng2 = jnp.where(c1, g1, ng)
        t1 = jnp.where(c1, nx, t1)
        g1 = jnp.where(c1, ng, g1)
        c2 = nx2 > t2
        t2 = jnp.where(c2, nx2, t2)
        g2 = jnp.where(c2, ng2, g2)
    t0_ref[...], t1_ref[...], t2_ref[...] = t0, t1, t2
    g0_ref[...], g1_ref[...], g2_ref[...] = g0, g1, g2

    @pl.when(kblk == NK - 1)
    def _():
        cand = jnp.concatenate([t0, t1, t2], axis=1)
        cidx = (jnp.concatenate([g0, g1, g2], axis=1) * 128
                + jnp.concatenate([liota, liota, liota], axis=1))
        ei = jax.lax.broadcasted_iota(jnp.int32, (BQ, K * 128), 1)
        vs, is_ = [], []
        for j in range(K):
            m = jnp.max(cand, axis=1, keepdims=True)
            p = jnp.min(jnp.where(cand == m, ei, K * 128), axis=1,
                        keepdims=True)
            onp = ei == p
            ii = jnp.max(jnp.where(onp, cidx, -1), axis=1, keepdims=True)
            vs.append(_pack(m))
            is_.append(_pack(ii))
            if j < K - 1:
                cand = jnp.where(onp, NEG, cand)
        ms = [jnp.where(v > REL_THRESH, v, 0.0) for v in vs]
        ssum = ms[0] + ms[1] + ms[2] + 1e-8
        for j in range(K):
            w_ref[j, :, :] = ms[j] / ssum
            idx_ref[j, :, :] = is_[j]


def _topk(q, kn16):
    return pl.pallas_call(
        _topk_kernel,
        grid=(NQ, NK),
        in_specs=[
            pl.BlockSpec((BQ, D), lambda i, k: (i, 0)),
            pl.BlockSpec((BK, D), lambda i, k: (k, 0)),
        ],
        out_specs=[
            pl.BlockSpec((K, PACK[0], 128), lambda i, k: (0, i, 0)),
            pl.BlockSpec((K, PACK[0], 128), lambda i, k: (0, i, 0)),
        ],
        out_shape=[
            jax.ShapeDtypeStruct((K, B // 128, 128), jnp.float32),
            jax.ShapeDtypeStruct((K, B // 128, 128), jnp.int32),
        ],
        scratch_shapes=[
            pltpu.VMEM((BQ, D), jnp.bfloat16),
            pltpu.VMEM((BQ, 128), jnp.float32),
            pltpu.VMEM((BQ, 128), jnp.float32),
            pltpu.VMEM((BQ, 128), jnp.float32),
            pltpu.VMEM((BQ, 128), jnp.int32),
            pltpu.VMEM((BQ, 128), jnp.int32),
            pltpu.VMEM((BQ, 128), jnp.int32),
        ],
    )(q, kn16)


_GATHER_WIN = 128


def _sc_gather(struct, idx_flat):
    n_idx = B * K

    @pl.kernel(
        out_type=jax.ShapeDtypeStruct((n_idx, S), jnp.float32),
        mesh=plsc.VectorSubcoreMesh(core_axis_name="core",
                                    subcore_axis_name="subcore"),
    )
    def _gather(x_hbm, i_hbm, o_hbm):
        def body(i_vmem, o_vmem):
            pltpu.sync_copy(x_hbm.at[i_vmem.at[0]], o_vmem)

        pltpu.emit_pipeline(
            body,
            grid=(n_idx // _GATHER_WIN,),
            in_specs=[pl.BlockSpec((1, _GATHER_WIN), lambda i: (0, i))],
            out_specs=[pl.BlockSpec((_GATHER_WIN, S), lambda i: (i, 0))],
            core_axis_name="subcore",
            dimension_semantics=(pltpu.PARALLEL,),
        )(i_hbm, o_hbm)

    return _gather(struct, idx_flat)


def _combine_kernel(w_ref, r0_ref, r1_ref, r2_ref, o_ref):
    w = w_ref[...]
    acc = (w[:, 0:1] * r0_ref[...] + w[:, 1:2] * r1_ref[...]
           + w[:, 2:3] * r2_ref[...])
    o_ref[...] = W_STRUCT * acc


def _combine(w, r0, r1, r2):
    bq = 1024
    return pl.pallas_call(
        _combine_kernel,
        grid=(B // bq,),
        in_specs=[
            pl.BlockSpec((bq, K), lambda i: (i, 0)),
            pl.BlockSpec((bq, S), lambda i: (i, 0)),
            pl.BlockSpec((bq, S), lambda i: (i, 0)),
            pl.BlockSpec((bq, S), lambda i: (i, 0)),
        ],
        out_specs=pl.BlockSpec((bq, S), lambda i: (i, 0)),
        out_shape=jax.ShapeDtypeStruct((B, S), jnp.float32),
    )(w, r0, r1, r2)


def kernel(sentence_embeddings, text_embeddings, structural_embeddings):
    kn16 = _normalize_keys(text_embeddings)
    wp, idxp = _topk(sentence_embeddings, kn16)
    idx_flat = idxp.reshape(1, K * B)
    retrieved = _sc_gather(structural_embeddings, idx_flat)
    r = retrieved.reshape(K, B, S)
    w = jnp.concatenate([wp[j].reshape(B, 1) for j in range(K)], axis=1)
    struct_ctx = _combine(w, r[0], r[1], r[2])
    return jnp.concatenate([sentence_embeddings, struct_ctx], axis=-1)
